# baseline (speedup 1.0000x reference)
import functools

import jax
import jax.numpy as jnp
from jax import lax
from jax.experimental import pallas as pl
from jax.experimental.pallas import tpu as pltpu
from jax.experimental.pallas import tpu_sc as plsc

N = 10000
D = 128
HH = 64
E = 320000
CHUNK = 128
NTILES = 16
CPT = 160
NCHP = NTILES * CPT
EC = NCHP * CHUNK
EPAD = EC - E
PN = 10240
RPT = PN // NTILES
BLK = 1000
F32 = jnp.float32


def _sc_agg_body(with_cnt, feat, ei, zeros, zeros1, *rest):
    if with_cnt:
        (out, cnt_out, idx_s, idx_d, rows, ones_v, acc, cnt_acc, sem) = rest
    else:
        (out, idx_s, idx_d, rows, acc, sem) = rest
    c = lax.axis_index("c")
    s = lax.axis_index("s")

    pltpu.sync_copy(zeros.at[pl.ds(s * RPT, RPT)], acc.at[pl.ds(s * RPT, RPT)])
    if with_cnt:
        pltpu.sync_copy(zeros1.at[pl.ds(s * RPT, RPT)],
                        cnt_acc.at[pl.ds(s * RPT, RPT)])
        for i in range(CHUNK // 16):
            ones_v[pl.ds(16 * i, 16)] = jnp.ones((16,), F32)
    plsc.subcore_barrier()

    def body(i, carry):
        eoff = c * EC + (s + NTILES * i) * CHUNK
        pltpu.sync_copy(ei.at[0, pl.ds(eoff, CHUNK)], idx_s)
        pltpu.sync_copy(ei.at[1, pl.ds(eoff, CHUNK)], idx_d)
        pltpu.async_copy(feat.at[idx_s], rows, sem).wait()
        pltpu.sync_copy(rows, acc.at[idx_d], add=True)
        if with_cnt:
            pltpu.sync_copy(ones_v, cnt_acc.at[idx_d], add=True)
        return carry

    lax.fori_loop(0, CPT, body, 0)

    plsc.subcore_barrier()
    pltpu.sync_copy(acc.at[pl.ds(s * RPT, RPT)], out.at[c, pl.ds(s * RPT, RPT)])
    if with_cnt:
        pltpu.sync_copy(cnt_acc.at[pl.ds(s * RPT, RPT)],
                        cnt_out.at[pl.ds(c * PN + s * RPT, RPT)])


@functools.lru_cache(maxsize=None)
def _make_sc_agg(with_cnt):
    out_type = [jax.ShapeDtypeStruct((2, PN, D), F32)]
    scratch = [
        pltpu.VMEM((CHUNK,), jnp.int32),
        pltpu.VMEM((CHUNK,), jnp.int32),
        pltpu.VMEM((CHUNK, D), F32),
        pltpu.VMEM_SHARED((PN, D), F32),
        pltpu.SemaphoreType.DMA,
    ]
    if with_cnt:
        out_type = [jax.ShapeDtypeStruct((2, PN, D), F32),
                    jax.ShapeDtypeStruct((2 * PN,), F32)]
        scratch = [
            pltpu.VMEM((CHUNK,), jnp.int32),
            pltpu.VMEM((CHUNK,), jnp.int32),
            pltpu.VMEM((CHUNK, D), F32),
            pltpu.VMEM((CHUNK,), F32),
            pltpu.VMEM_SHARED((PN, D), F32),
            pltpu.VMEM_SHARED((PN,), F32),
            pltpu.SemaphoreType.DMA,
        ]
    return pl.kernel(
        functools.partial(_sc_agg_body, with_cnt),
        out_type=out_type,
        mesh=plsc.VectorSubcoreMesh(core_axis_name="c", subcore_axis_name="s"),
        scratch_types=scratch,
    )


def _tc1_body(emb_ref, sp_ref, sn_ref, cp_ref, cn_ref, wpl_ref, wnl_ref,
              ws_ref, b_ref, lnw_ref, lnb_ref, out_ref):
    emb = emb_ref[...]
    inv_p = 1.0 / jnp.maximum(cp_ref[0], 1.0)
    inv_n = 1.0 / jnp.maximum(cn_ref[0], 1.0)
    ap = sp_ref[0] * inv_p
    an = sn_ref[0] * inv_n
    mp = jnp.dot(ap, wpl_ref[...], preferred_element_type=F32)
    mn = jnp.dot(an, wnl_ref[...], preferred_element_type=F32)
    ms = jnp.dot(emb, ws_ref[...], preferred_element_type=F32)
    x = jnp.concatenate([mp, mn], axis=1) + ms + b_ref[...]
    x = jnp.maximum(x, 0.0)
    mu = jnp.mean(x, axis=1, keepdims=True)
    xc = x - mu
    var = jnp.mean(xc * xc, axis=1, keepdims=True)
    out_ref[...] = xc * lax.rsqrt(var + 1e-5) * lnw_ref[...] + lnb_ref[...]


_tc1 = pl.pallas_call(
    _tc1_body,
    grid=(N // BLK,),
    in_specs=[
        pl.BlockSpec((BLK, D), lambda i: (i, 0)),
        pl.BlockSpec((1, BLK, D), lambda i: (0, i, 0)),
        pl.BlockSpec((1, BLK, D), lambda i: (1, i, 0)),
        pl.BlockSpec((1, BLK, 1), lambda i: (0, i, 0)),
        pl.BlockSpec((1, BLK, 1), lambda i: (1, i, 0)),
        pl.BlockSpec((D, HH), lambda i: (0, 0)),
        pl.BlockSpec((D, HH), lambda i: (0, 0)),
        pl.BlockSpec((D, D), lambda i: (0, 0)),
        pl.BlockSpec((1, D), lambda i: (0, 0)),
        pl.BlockSpec((1, D), lambda i: (0, 0)),
        pl.BlockSpec((1, D), lambda i: (0, 0)),
    ],
    out_specs=pl.BlockSpec((BLK, D), lambda i: (i, 0)),
    out_shape=jax.ShapeDtypeStruct((N, D), F32),
)


def _tc2_body(x_ref, emb_ref, s2p_ref, s2n_ref, cp_ref, cn_ref, w2pl_ref,
              w2nl_ref, w2pr_ref, w2nr_ref, b2_ref, ln2w_ref, ln2b_ref,
              wh1_ref, bh1_ref, wh2_ref, bh2_ref, gate_ref, out_ref):
    x = x_ref[...]
    inv_p = 1.0 / jnp.maximum(cp_ref[0], 1.0)
    inv_n = 1.0 / jnp.maximum(cn_ref[0], 1.0)
    mp = s2p_ref[0] * inv_p
    mn = s2n_ref[0] * inv_n
    lhs_p = jnp.concatenate([mp[:, :HH], mn[:, HH:D]], axis=1)
    lhs_n = jnp.concatenate([mp[:, HH:D], mn[:, :HH]], axis=1)
    op = (jnp.dot(lhs_p, w2pl_ref[...], preferred_element_type=F32)
          + jnp.dot(x[:, :HH], w2pr_ref[...], preferred_element_type=F32))
    on = (jnp.dot(lhs_n, w2nl_ref[...], preferred_element_type=F32)
          + jnp.dot(x[:, HH:], w2nr_ref[...], preferred_element_type=F32))
    x2 = jnp.concatenate([op, on], axis=1) + b2_ref[...]
    x2 = jnp.maximum(x2, 0.0)
    mu = jnp.mean(x2, axis=1, keepdims=True)
    xc = x2 - mu
    var = jnp.mean(xc * xc, axis=1, keepdims=True)
    x2 = xc * lax.rsqrt(var + 1e-5) * ln2w_ref[...] + ln2b_ref[...]
    h = jnp.dot(x2, wh1_ref[...], preferred_element_type=F32) + bh1_ref[...]
    h = 0.5 * h * (1.0 + lax.erf(h * 0.7071067811865476))
    h = jnp.dot(h, wh2_ref[...], preferred_element_type=F32) + bh2_ref[...]
    g = gate_ref[0, 0]
    out_ref[...] = g * h + (1.0 - g) * emb_ref[...]


_tc2 = pl.pallas_call(
    _tc2_body,
    grid=(N // BLK,),
    in_specs=[
        pl.BlockSpec((BLK, D), lambda i: (i, 0)),
        pl.BlockSpec((BLK, D), lambda i: (i, 0)),
        pl.BlockSpec((1, BLK, D), lambda i: (0, i, 0)),
        pl.BlockSpec((1, BLK, D), lambda i: (1, i, 0)),
        pl.BlockSpec((1, BLK, 1), lambda i: (0, i, 0)),
        pl.BlockSpec((1, BLK, 1), lambda i: (1, i, 0)),
        pl.BlockSpec((D, HH), lambda i: (0, 0)),
        pl.BlockSpec((D, HH), lambda i: (0, 0)),
        pl.BlockSpec((HH, HH), lambda i: (0, 0)),
        pl.BlockSpec((HH, HH), lambda i: (0, 0)),
        pl.BlockSpec((1, D), lambda i: (0, 0)),
        pl.BlockSpec((1, D), lambda i: (0, 0)),
        pl.BlockSpec((1, D), lambda i: (0, 0)),
        pl.BlockSpec((D, D), lambda i: (0, 0)),
        pl.BlockSpec((1, D), lambda i: (0, 0)),
        pl.BlockSpec((D, D), lambda i: (0, 0)),
        pl.BlockSpec((1, D), lambda i: (0, 0)),
        pl.BlockSpec((1, 1), lambda i: (0, 0)),
    ],
    out_specs=pl.BlockSpec((BLK, D), lambda i: (i, 0)),
    out_shape=jax.ShapeDtypeStruct((N, D), F32),
)


def kernel(pretrained_emb, pos_edge_index, neg_edge_index,
           c1_pl_w, c1_pl_b, c1_pr_w, c1_pr_b, c1_nl_w, c1_nl_b,
           c1_nr_w, c1_nr_b, c2_pl_w, c2_pl_b, c2_pr_w, c2_pr_b,
           c2_nl_w, c2_nl_b, c2_nr_w, c2_nr_b, ln1_w, ln1_b,
           ln2_w, ln2_b, ph1_w, ph1_b, ph2_w, ph2_b, gate):
    emb = pretrained_emb.astype(F32)
    ar = jnp.arange(EPAD, dtype=jnp.int32)
    pad = jnp.concatenate(
        [(ar % N).reshape(1, EPAD),
         (N + ar % (PN - N)).reshape(1, EPAD)], axis=0)
    eis = jnp.concatenate([pos_edge_index, pad, neg_edge_index, pad], axis=1)
    znd = jnp.zeros((PN, D), F32)
    zn1 = jnp.zeros((PN,), F32)

    sums1, cnt = _make_sc_agg(True)(emb, eis, znd, zn1)
    cnt3 = cnt.reshape(2, PN, 1)

    b1 = jnp.concatenate([c1_pl_b + c1_pr_b, c1_nl_b + c1_nr_b]).reshape(1, D)
    ws1 = jnp.concatenate([c1_pr_w.T, c1_nr_w.T], axis=1)
    x = _tc1(emb, sums1, sums1, cnt3, cnt3, c1_pl_w.T, c1_nl_w.T, ws1, b1,
             ln1_w.reshape(1, D), ln1_b.reshape(1, D))

    (sums2,) = _make_sc_agg(False)(x, eis, znd, zn1)

    b2 = jnp.concatenate([c2_pl_b + c2_pr_b, c2_nl_b + c2_nr_b]).reshape(1, D)
    out = _tc2(x, emb, sums2, sums2, cnt3, cnt3, c2_pl_w.T, c2_nl_w.T,
               c2_pr_w.T, c2_nr_w.T, b2, ln2_w.reshape(1, D),
               ln2_b.reshape(1, D), ph1_w.T, ph1_b.reshape(1, D), ph2_w.T,
               ph2_b.reshape(1, D), jnp.asarray(gate, F32).reshape(1, 1))
    return out


# speedup vs baseline: 1.6237x; 1.6237x over previous
import functools

import jax
import jax.numpy as jnp
from jax import lax
from jax.experimental import pallas as pl
from jax.experimental.pallas import tpu as pltpu
from jax.experimental.pallas import tpu_sc as plsc

N = 10000
D = 128
HH = 64
E = 320000
CHUNK = 128
NTILES = 16
CPT = 160
NCHP = NTILES * CPT
EC = NCHP * CHUNK
EPAD = EC - E
PN = 10240
RPT = PN // NTILES
BLK = 1000
F32 = jnp.float32


def _sc_agg_body(with_cnt, feat, ei, zeros, zeros1, *rest):
    if with_cnt:
        (out, cnt_out, idx_s, idx_d, rows0, rows1, ones_v, acc, cnt_acc,
         sem0, sem1) = rest
    else:
        (out, idx_s, idx_d, rows0, rows1, acc, sem0, sem1) = rest
    c = lax.axis_index("c")
    s = lax.axis_index("s")

    pltpu.sync_copy(zeros.at[pl.ds(s * RPT, RPT)], acc.at[pl.ds(s * RPT, RPT)])
    if with_cnt:
        pltpu.sync_copy(zeros1.at[pl.ds(s * RPT, RPT)],
                        cnt_acc.at[pl.ds(s * RPT, RPT)])
        for i in range(CHUNK // 16):
            ones_v[pl.ds(16 * i, 16)] = jnp.ones((16,), F32)
    plsc.subcore_barrier()

    rows = (rows0, rows1)
    sems = (sem0, sem1)

    def start_g(k, kb):
        pltpu.async_copy(feat.at[idx_s.at[k]], rows[kb], sems[kb])

    def wait_g(kb):
        pltpu.make_async_copy(feat.at[idx_s.at[0]], rows[kb], sems[kb]).wait()

    def scat(k, kb):
        pltpu.sync_copy(rows[kb], acc.at[idx_d.at[k]], add=True)
        if with_cnt:
            pltpu.sync_copy(ones_v, cnt_acc.at[idx_d.at[k]], add=True)

    base = c * NCHP + s * CPT
    def body(oi, carry):
        j0 = base + 8 * oi
        pltpu.sync_copy(ei.at[0, pl.ds(j0, 8)], idx_s)
        pltpu.sync_copy(ei.at[1, pl.ds(j0, 8)], idx_d)
        start_g(0, 0)
        for k in range(7):
            wait_g(k % 2)
            start_g(k + 1, (k + 1) % 2)
            scat(k, k % 2)
        wait_g(1)
        scat(7, 1)
        return carry

    lax.fori_loop(0, CPT // 8, body, 0)

    plsc.subcore_barrier()
    pltpu.sync_copy(acc.at[pl.ds(s * RPT, RPT)], out.at[c, pl.ds(s * RPT, RPT)])
    if with_cnt:
        pltpu.sync_copy(cnt_acc.at[pl.ds(s * RPT, RPT)],
                        cnt_out.at[pl.ds(c * PN + s * RPT, RPT)])


@functools.lru_cache(maxsize=None)
def _make_sc_agg(with_cnt):
    out_type = [jax.ShapeDtypeStruct((2, PN, D), F32)]
    scratch = [
        pltpu.VMEM((8, CHUNK), jnp.int32),
        pltpu.VMEM((8, CHUNK), jnp.int32),
        pltpu.VMEM((CHUNK, D), F32),
        pltpu.VMEM((CHUNK, D), F32),
        pltpu.VMEM_SHARED((PN, D), F32),
        pltpu.SemaphoreType.DMA,
        pltpu.SemaphoreType.DMA,
    ]
    if with_cnt:
        out_type = [jax.ShapeDtypeStruct((2, PN, D), F32),
                    jax.ShapeDtypeStruct((2 * PN,), F32)]
        scratch = [
            pltpu.VMEM((8, CHUNK), jnp.int32),
            pltpu.VMEM((8, CHUNK), jnp.int32),
            pltpu.VMEM((CHUNK, D), F32),
            pltpu.VMEM((CHUNK, D), F32),
            pltpu.VMEM((CHUNK,), F32),
            pltpu.VMEM_SHARED((PN, D), F32),
            pltpu.VMEM_SHARED((PN,), F32),
            pltpu.SemaphoreType.DMA,
            pltpu.SemaphoreType.DMA,
        ]
    return pl.kernel(
        functools.partial(_sc_agg_body, with_cnt),
        out_type=out_type,
        mesh=plsc.VectorSubcoreMesh(core_axis_name="c", subcore_axis_name="s"),
        scratch_types=scratch,
    )


def _tc1_body(emb_ref, sp_ref, sn_ref, cp_ref, cn_ref, wpl_ref, wnl_ref,
              ws_ref, b_ref, lnw_ref, lnb_ref, out_ref):
    emb = emb_ref[...]
    inv_p = 1.0 / jnp.maximum(cp_ref[0], 1.0)
    inv_n = 1.0 / jnp.maximum(cn_ref[0], 1.0)
    ap = sp_ref[0] * inv_p
    an = sn_ref[0] * inv_n
    mp = jnp.dot(ap, wpl_ref[...], preferred_element_type=F32)
    mn = jnp.dot(an, wnl_ref[...], preferred_element_type=F32)
    ms = jnp.dot(emb, ws_ref[...], preferred_element_type=F32)
    x = jnp.concatenate([mp, mn], axis=1) + ms + b_ref[...]
    x = jnp.maximum(x, 0.0)
    mu = jnp.mean(x, axis=1, keepdims=True)
    xc = x - mu
    var = jnp.mean(xc * xc, axis=1, keepdims=True)
    out_ref[...] = xc * lax.rsqrt(var + 1e-5) * lnw_ref[...] + lnb_ref[...]


_tc1 = pl.pallas_call(
    _tc1_body,
    grid=(N // BLK,),
    in_specs=[
        pl.BlockSpec((BLK, D), lambda i: (i, 0)),
        pl.BlockSpec((1, BLK, D), lambda i: (0, i, 0)),
        pl.BlockSpec((1, BLK, D), lambda i: (1, i, 0)),
        pl.BlockSpec((1, BLK, 1), lambda i: (0, i, 0)),
        pl.BlockSpec((1, BLK, 1), lambda i: (1, i, 0)),
        pl.BlockSpec((D, HH), lambda i: (0, 0)),
        pl.BlockSpec((D, HH), lambda i: (0, 0)),
        pl.BlockSpec((D, D), lambda i: (0, 0)),
        pl.BlockSpec((1, D), lambda i: (0, 0)),
        pl.BlockSpec((1, D), lambda i: (0, 0)),
        pl.BlockSpec((1, D), lambda i: (0, 0)),
    ],
    out_specs=pl.BlockSpec((BLK, D), lambda i: (i, 0)),
    out_shape=jax.ShapeDtypeStruct((N, D), F32),
)


def _tc2_body(x_ref, emb_ref, s2p_ref, s2n_ref, cp_ref, cn_ref, w2pl_ref,
              w2nl_ref, w2pr_ref, w2nr_ref, b2_ref, ln2w_ref, ln2b_ref,
              wh1_ref, bh1_ref, wh2_ref, bh2_ref, gate_ref, out_ref):
    x = x_ref[...]
    inv_p = 1.0 / jnp.maximum(cp_ref[0], 1.0)
    inv_n = 1.0 / jnp.maximum(cn_ref[0], 1.0)
    mp = s2p_ref[0] * inv_p
    mn = s2n_ref[0] * inv_n
    lhs_p = jnp.concatenate([mp[:, :HH], mn[:, HH:D]], axis=1)
    lhs_n = jnp.concatenate([mp[:, HH:D], mn[:, :HH]], axis=1)
    op = (jnp.dot(lhs_p, w2pl_ref[...], preferred_element_type=F32)
          + jnp.dot(x[:, :HH], w2pr_ref[...], preferred_element_type=F32))
    on = (jnp.dot(lhs_n, w2nl_ref[...], preferred_element_type=F32)
          + jnp.dot(x[:, HH:], w2nr_ref[...], preferred_element_type=F32))
    x2 = jnp.concatenate([op, on], axis=1) + b2_ref[...]
    x2 = jnp.maximum(x2, 0.0)
    mu = jnp.mean(x2, axis=1, keepdims=True)
    xc = x2 - mu
    var = jnp.mean(xc * xc, axis=1, keepdims=True)
    x2 = xc * lax.rsqrt(var + 1e-5) * ln2w_ref[...] + ln2b_ref[...]
    h = jnp.dot(x2, wh1_ref[...], preferred_element_type=F32) + bh1_ref[...]
    h = 0.5 * h * (1.0 + lax.erf(h * 0.7071067811865476))
    h = jnp.dot(h, wh2_ref[...], preferred_element_type=F32) + bh2_ref[...]
    g = gate_ref[0, 0]
    out_ref[...] = g * h + (1.0 - g) * emb_ref[...]


_tc2 = pl.pallas_call(
    _tc2_body,
    grid=(N // BLK,),
    in_specs=[
        pl.BlockSpec((BLK, D), lambda i: (i, 0)),
        pl.BlockSpec((BLK, D), lambda i: (i, 0)),
        pl.BlockSpec((1, BLK, D), lambda i: (0, i, 0)),
        pl.BlockSpec((1, BLK, D), lambda i: (1, i, 0)),
        pl.BlockSpec((1, BLK, 1), lambda i: (0, i, 0)),
        pl.BlockSpec((1, BLK, 1), lambda i: (1, i, 0)),
        pl.BlockSpec((D, HH), lambda i: (0, 0)),
        pl.BlockSpec((D, HH), lambda i: (0, 0)),
        pl.BlockSpec((HH, HH), lambda i: (0, 0)),
        pl.BlockSpec((HH, HH), lambda i: (0, 0)),
        pl.BlockSpec((1, D), lambda i: (0, 0)),
        pl.BlockSpec((1, D), lambda i: (0, 0)),
        pl.BlockSpec((1, D), lambda i: (0, 0)),
        pl.BlockSpec((D, D), lambda i: (0, 0)),
        pl.BlockSpec((1, D), lambda i: (0, 0)),
        pl.BlockSpec((D, D), lambda i: (0, 0)),
        pl.BlockSpec((1, D), lambda i: (0, 0)),
        pl.BlockSpec((1, 1), lambda i: (0, 0)),
    ],
    out_specs=pl.BlockSpec((BLK, D), lambda i: (i, 0)),
    out_shape=jax.ShapeDtypeStruct((N, D), F32),
)


def kernel(pretrained_emb, pos_edge_index, neg_edge_index,
           c1_pl_w, c1_pl_b, c1_pr_w, c1_pr_b, c1_nl_w, c1_nl_b,
           c1_nr_w, c1_nr_b, c2_pl_w, c2_pl_b, c2_pr_w, c2_pr_b,
           c2_nl_w, c2_nl_b, c2_nr_w, c2_nr_b, ln1_w, ln1_b,
           ln2_w, ln2_b, ph1_w, ph1_b, ph2_w, ph2_b, gate):
    emb = pretrained_emb.astype(F32)
    ar = jnp.arange(EPAD, dtype=jnp.int32)
    pad = jnp.concatenate(
        [(ar % N).reshape(1, EPAD),
         (N + ar % (PN - N)).reshape(1, EPAD)], axis=0)
    eis = jnp.concatenate(
        [pos_edge_index, pad, neg_edge_index, pad], axis=1
    ).reshape(2, 2 * NCHP, CHUNK)
    znd = jnp.zeros((PN, D), F32)
    zn1 = jnp.zeros((PN,), F32)

    sums1, cnt = _make_sc_agg(True)(emb, eis, znd, zn1)
    cnt3 = cnt.reshape(2, PN, 1)

    b1 = jnp.concatenate([c1_pl_b + c1_pr_b, c1_nl_b + c1_nr_b]).reshape(1, D)
    ws1 = jnp.concatenate([c1_pr_w.T, c1_nr_w.T], axis=1)
    x = _tc1(emb, sums1, sums1, cnt3, cnt3, c1_pl_w.T, c1_nl_w.T, ws1, b1,
             ln1_w.reshape(1, D), ln1_b.reshape(1, D))

    (sums2,) = _make_sc_agg(False)(x, eis, znd, zn1)

    b2 = jnp.concatenate([c2_pl_b + c2_pr_b, c2_nl_b + c2_nr_b]).reshape(1, D)
    out = _tc2(x, emb, sums2, sums2, cnt3, cnt3, c2_pl_w.T, c2_nl_w.T,
               c2_pr_w.T, c2_nr_w.T, b2, ln2_w.reshape(1, D),
               ln2_b.reshape(1, D), ph1_w.T, ph1_b.reshape(1, D), ph2_w.T,
               ph2_b.reshape(1, D), jnp.asarray(gate, F32).reshape(1, 1))
    return out
